# baseline (speedup 1.0000x reference)
import jax
import jax.numpy as jnp
from jax import lax
from jax.experimental import pallas as pl
from jax.experimental.pallas import tpu as pltpu
from jax.experimental.pallas import tpu_sc as plsc

N_NODES = 50000
N_EDGES = 1600000
D_NODE = 256
D_EDGE = 16
D_GLOBAL = 16

NC = 2
NS = 16
NW = NC * NS

IDX_ROW = 128
N_ROWS = N_EDGES // IDX_ROW
ROWS_BASE = N_ROWS // NW
ROWS_EXTRA = N_ROWS - ROWS_BASE * NW

GROUP_ROWS = 15
GROUP_EDGES = GROUP_ROWS * IDX_ROW
N_GROUPS = ROWS_BASE // GROUP_ROWS

ACC_ROWS = 50048
SLICE = ACC_ROWS // NS


def _sc_body(edges_hbm, dst_hbm, zeros_hbm, zeros16_hbm, ones_hbm,
             psum_hbm, pcnt_hbm, acc_sum, acc_cnt, ebuf, ibuf, ones_v):
    c = lax.axis_index("c")
    s = lax.axis_index("s")
    w = c * NS + s

    pltpu.sync_copy(zeros_hbm, acc_sum.at[pl.ds(s * SLICE, SLICE)])
    pltpu.sync_copy(zeros16_hbm, acc_cnt.at[pl.ds(s * SLICE, SLICE)])
    pltpu.sync_copy(ones_hbm, ones_v)
    plsc.subcore_barrier()

    row_start = ROWS_BASE * w + jnp.minimum(w, ROWS_EXTRA)

    def stage(r0, n_rows):
        for f in range(D_EDGE):
            pltpu.sync_copy(
                edges_hbm.at[f // 8, pl.ds(r0, n_rows), f % 8],
                ebuf.at[pl.ds(0, n_rows), :, f])
        pltpu.sync_copy(dst_hbm.at[pl.ds(r0, n_rows), 1],
                        ibuf.at[pl.ds(0, n_rows)])

    def group(g, carry):
        r0 = row_start + g * GROUP_ROWS
        stage(r0, GROUP_ROWS)
        for j in range(GROUP_ROWS):
            pltpu.sync_copy(ebuf.at[j], acc_sum.at[ibuf.at[j]], add=True)
            pltpu.sync_copy(ones_v, acc_cnt.at[ibuf.at[j]], add=True)
        return carry

    lax.fori_loop(0, N_GROUPS, group, 0)

    @pl.when(w < ROWS_EXTRA)
    def _():
        r0 = row_start + ROWS_BASE
        stage(r0, 1)
        pltpu.sync_copy(ebuf.at[0], acc_sum.at[ibuf.at[0]], add=True)
        pltpu.sync_copy(ones_v, acc_cnt.at[ibuf.at[0]], add=True)

    plsc.subcore_barrier()

    pltpu.sync_copy(acc_sum.at[pl.ds(s * SLICE, SLICE)],
                    psum_hbm.at[c, pl.ds(s * SLICE, SLICE)])
    pltpu.sync_copy(acc_cnt.at[pl.ds(s * SLICE, SLICE)],
                    pcnt_hbm.at[c, pl.ds(s * SLICE, SLICE)])


_sc_aggregate = pl.kernel(
    _sc_body,
    out_type=(jax.ShapeDtypeStruct((NC, ACC_ROWS, D_EDGE), jnp.float32),
              jax.ShapeDtypeStruct((NC, ACC_ROWS, D_EDGE), jnp.int16)),
    mesh=plsc.VectorSubcoreMesh(core_axis_name="c", subcore_axis_name="s",
                                num_cores=NC, num_subcores=NS),
    scratch_types=[
        pltpu.VMEM_SHARED((ACC_ROWS, D_EDGE), jnp.float32),
        pltpu.VMEM_SHARED((ACC_ROWS, D_EDGE), jnp.int16),
        pltpu.VMEM((GROUP_ROWS, IDX_ROW, D_EDGE), jnp.float32),
        pltpu.VMEM((GROUP_ROWS, IDX_ROW), jnp.int32),
        pltpu.VMEM((IDX_ROW, D_EDGE), jnp.int16),
    ],
    compiler_params=pltpu.CompilerParams(use_tc_tiling_on_sc=False),
)

BN = 400


def _tc_body(psum_ref, pcnt_ref, nodes_ref, g_ref, out_ref):
    sums = psum_ref[0] + psum_ref[1]
    cnts = (pcnt_ref[0].astype(jnp.float32) +
            pcnt_ref[1].astype(jnp.float32))
    mean = sums / jnp.maximum(cnts, 1.0)
    g = jnp.broadcast_to(g_ref[...], (BN, D_GLOBAL))
    out_ref[...] = jnp.concatenate([mean, nodes_ref[...], g], axis=1)


def kernel(nodes_data, edges_data, global_data, edge_index):
    edges_b = (edges_data.reshape(N_ROWS, IDX_ROW, 2, 8)
               .transpose(2, 0, 3, 1))
    dst_b = (edge_index.astype(jnp.int32)
             .reshape(2, N_ROWS, IDX_ROW).transpose(1, 0, 2))
    zeros_blk = jnp.zeros((SLICE, D_EDGE), jnp.float32)
    zeros_blk_i16 = jnp.zeros((SLICE, D_EDGE), jnp.int16)
    ones_blk = jnp.ones((IDX_ROW, D_EDGE), jnp.int16)

    psum, pcnt = _sc_aggregate(edges_b, dst_b, zeros_blk, zeros_blk_i16,
                               ones_blk)

    out = pl.pallas_call(
        _tc_body,
        grid=(N_NODES // BN,),
        in_specs=[
            pl.BlockSpec((NC, BN, D_EDGE), lambda i: (0, i, 0)),
            pl.BlockSpec((NC, BN, D_EDGE), lambda i: (0, i, 0)),
            pl.BlockSpec((BN, D_NODE), lambda i: (i, 0)),
            pl.BlockSpec((1, D_GLOBAL), lambda i: (0, 0)),
        ],
        out_specs=pl.BlockSpec((BN, D_NODE + 2 * D_EDGE), lambda i: (i, 0)),
        out_shape=jax.ShapeDtypeStruct((N_NODES, D_NODE + 2 * D_EDGE),
                                       jnp.float32),
    )(psum, pcnt, nodes_data, global_data.reshape(1, D_GLOBAL))
    return out


# speedup vs baseline: 3.3209x; 3.3209x over previous
import jax
import jax.numpy as jnp
from jax import lax
from jax.experimental import pallas as pl
from jax.experimental.pallas import tpu as pltpu
from jax.experimental.pallas import tpu_sc as plsc

N_NODES = 50000
N_EDGES = 1600000
D_NODE = 256
D_EDGE = 16
D_GLOBAL = 16

NC = 2
NS = 16
NW = NC * NS

IDX_ROW = 128
N_ROWS = N_EDGES // IDX_ROW
ROWS_BASE = N_ROWS // NW
ROWS_EXTRA = N_ROWS - ROWS_BASE * NW

GROUP_ROWS = 15
GROUP_EDGES = GROUP_ROWS * IDX_ROW
N_GROUPS = ROWS_BASE // GROUP_ROWS

ACC_ROWS = 50048
SLICE = ACC_ROWS // NS


def _sc_body(edges_hbm, dst_hbm, zeros_hbm, zeros16_hbm, ones_hbm,
             psum_hbm, pcnt_hbm, acc_sum, acc_cnt, ebuf, ibuf, ones_v):
    c = lax.axis_index("c")
    s = lax.axis_index("s")
    w = c * NS + s

    pltpu.sync_copy(zeros_hbm, acc_sum.at[pl.ds(s * SLICE, SLICE)])
    pltpu.sync_copy(zeros16_hbm, acc_cnt.at[pl.ds(s * SLICE, SLICE)])
    pltpu.sync_copy(ones_hbm, ones_v)
    plsc.subcore_barrier()

    row_start = ROWS_BASE * w + jnp.minimum(w, ROWS_EXTRA)

    def stage(r0, n_rows):
        pltpu.sync_copy(edges_hbm.at[pl.ds(r0 * IDX_ROW, n_rows * IDX_ROW)],
                        ebuf.at[pl.ds(0, n_rows * IDX_ROW)])
        pltpu.sync_copy(dst_hbm.at[pl.ds(r0, n_rows), 1],
                        ibuf.at[pl.ds(0, n_rows)])

    def group(g, carry):
        r0 = row_start + g * GROUP_ROWS
        stage(r0, GROUP_ROWS)
        for j in range(GROUP_ROWS):
            pltpu.sync_copy(ebuf.at[pl.ds(j * IDX_ROW, IDX_ROW)],
                            acc_sum.at[ibuf.at[j]], add=True)
            pltpu.sync_copy(ones_v, acc_cnt.at[ibuf.at[j]], add=True)
        return carry

    lax.fori_loop(0, N_GROUPS, group, 0)

    @pl.when(w < ROWS_EXTRA)
    def _():
        r0 = row_start + ROWS_BASE
        stage(r0, 1)
        pltpu.sync_copy(ebuf.at[pl.ds(0, IDX_ROW)],
                        acc_sum.at[ibuf.at[0]], add=True)
        pltpu.sync_copy(ones_v, acc_cnt.at[ibuf.at[0]], add=True)

    plsc.subcore_barrier()

    pltpu.sync_copy(acc_sum.at[pl.ds(s * SLICE, SLICE)],
                    psum_hbm.at[c, pl.ds(s * SLICE, SLICE)])
    pltpu.sync_copy(acc_cnt.at[pl.ds(s * SLICE, SLICE)],
                    pcnt_hbm.at[c, pl.ds(s * SLICE, SLICE)])


_sc_aggregate = pl.kernel(
    _sc_body,
    out_type=(jax.ShapeDtypeStruct((NC, ACC_ROWS, D_EDGE), jnp.float32),
              jax.ShapeDtypeStruct((NC, ACC_ROWS, D_EDGE), jnp.int16)),
    mesh=plsc.VectorSubcoreMesh(core_axis_name="c", subcore_axis_name="s",
                                num_cores=NC, num_subcores=NS),
    scratch_types=[
        pltpu.VMEM_SHARED((ACC_ROWS, D_EDGE), jnp.float32),
        pltpu.VMEM_SHARED((ACC_ROWS, D_EDGE), jnp.int16),
        pltpu.VMEM((GROUP_EDGES, D_EDGE), jnp.float32),
        pltpu.VMEM((GROUP_ROWS, IDX_ROW), jnp.int32),
        pltpu.VMEM((IDX_ROW, D_EDGE), jnp.int16),
    ],
    compiler_params=pltpu.CompilerParams(use_tc_tiling_on_sc=False),
)

BN = 400


def _tc_body(psum_ref, pcnt_ref, nodes_ref, g_ref, out_ref):
    sums = psum_ref[0] + psum_ref[1]
    cnts = (pcnt_ref[0].astype(jnp.float32) +
            pcnt_ref[1].astype(jnp.float32))
    mean = sums / jnp.maximum(cnts, 1.0)
    g = jnp.broadcast_to(g_ref[...], (BN, D_GLOBAL))
    out_ref[...] = jnp.concatenate([mean, nodes_ref[...], g], axis=1)


def kernel(nodes_data, edges_data, global_data, edge_index):
    dst_b = (edge_index.astype(jnp.int32)
             .reshape(2, N_ROWS, IDX_ROW).transpose(1, 0, 2))
    zeros_blk = jnp.zeros((SLICE, D_EDGE), jnp.float32)
    zeros_blk_i16 = jnp.zeros((SLICE, D_EDGE), jnp.int16)
    ones_blk = jnp.ones((IDX_ROW, D_EDGE), jnp.int16)

    psum, pcnt = _sc_aggregate(edges_data, dst_b, zeros_blk, zeros_blk_i16,
                               ones_blk)

    out = pl.pallas_call(
        _tc_body,
        grid=(N_NODES // BN,),
        in_specs=[
            pl.BlockSpec((NC, BN, D_EDGE), lambda i: (0, i, 0)),
            pl.BlockSpec((NC, BN, D_EDGE), lambda i: (0, i, 0)),
            pl.BlockSpec((BN, D_NODE), lambda i: (i, 0)),
            pl.BlockSpec((1, D_GLOBAL), lambda i: (0, 0)),
        ],
        out_specs=pl.BlockSpec((BN, D_NODE + 2 * D_EDGE), lambda i: (i, 0)),
        out_shape=jax.ShapeDtypeStruct((N_NODES, D_NODE + 2 * D_EDGE),
                                       jnp.float32),
    )(psum, pcnt, nodes_data, global_data.reshape(1, D_GLOBAL))
    return out


# speedup vs baseline: 3.6010x; 1.0843x over previous
import jax
import jax.numpy as jnp
from jax import lax
from jax.experimental import pallas as pl
from jax.experimental.pallas import tpu as pltpu
from jax.experimental.pallas import tpu_sc as plsc

N_NODES = 50000
N_EDGES = 1600000
D_NODE = 256
D_EDGE = 16
D_GLOBAL = 16

NC = 2
NS = 16
NW = NC * NS

IDX_ROW = 128
N_ROWS = N_EDGES // IDX_ROW
ROWS_BASE = N_ROWS // NW
ROWS_EXTRA = N_ROWS - ROWS_BASE * NW

GROUP_ROWS = 10
GROUP_EDGES = GROUP_ROWS * IDX_ROW
N_GROUPS = ROWS_BASE // GROUP_ROWS

ACC_ROWS = 50048
SLICE = ACC_ROWS // NS


def _sc_body(edges_hbm, dst_hbm, zeros_hbm, zeros16_hbm, ones_hbm,
             psum_hbm, pcnt_hbm, acc_sum, acc_cnt, ebuf, ebuf_t, ibuf,
             ones_v):
    c = lax.axis_index("c")
    s = lax.axis_index("s")
    w = c * NS + s

    pltpu.sync_copy(zeros_hbm, acc_sum.at[pl.ds(s * SLICE, SLICE)])
    pltpu.sync_copy(zeros16_hbm, acc_cnt.at[pl.ds(s * SLICE, SLICE)])
    pltpu.sync_copy(ones_hbm, ones_v)
    plsc.subcore_barrier()

    row_start = ROWS_BASE * w + jnp.minimum(w, ROWS_EXTRA)

    iota16 = lax.iota(jnp.int32, 16)

    def stage(r0, n_rows):
        for f in range(D_EDGE):
            pltpu.sync_copy(edges_hbm.at[f // 8, pl.ds(r0, n_rows), f % 8],
                            ebuf_t.at[f, pl.ds(0, n_rows)])
        pltpu.sync_copy(dst_hbm.at[pl.ds(r0, n_rows), 1],
                        ibuf.at[pl.ds(0, n_rows)])

        def irow(j, carry):
            for lc in range(IDX_ROW // 16):
                rows = j * IDX_ROW + lc * 16 + iota16
                for f in range(D_EDGE):
                    v = ebuf_t[f, j, pl.ds(lc * 16, 16)]
                    plsc.store_scatter(
                        ebuf, [rows, jnp.full((16,), f, jnp.int32)], v)
            return carry
        lax.fori_loop(0, n_rows, irow, 0)

    def group(g, carry):
        r0 = row_start + g * GROUP_ROWS
        stage(r0, GROUP_ROWS)
        for j in range(GROUP_ROWS):
            pltpu.sync_copy(ebuf.at[pl.ds(j * IDX_ROW, IDX_ROW)],
                            acc_sum.at[ibuf.at[j]], add=True)
            pltpu.sync_copy(ones_v, acc_cnt.at[ibuf.at[j]], add=True)
        return carry

    lax.fori_loop(0, N_GROUPS, group, 0)

    @pl.when(w < ROWS_EXTRA)
    def _():
        r0 = row_start + ROWS_BASE
        stage(r0, 1)
        pltpu.sync_copy(ebuf.at[pl.ds(0, IDX_ROW)],
                        acc_sum.at[ibuf.at[0]], add=True)
        pltpu.sync_copy(ones_v, acc_cnt.at[ibuf.at[0]], add=True)

    plsc.subcore_barrier()

    pltpu.sync_copy(acc_sum.at[pl.ds(s * SLICE, SLICE)],
                    psum_hbm.at[c, pl.ds(s * SLICE, SLICE)])
    pltpu.sync_copy(acc_cnt.at[pl.ds(s * SLICE, SLICE)],
                    pcnt_hbm.at[c, pl.ds(s * SLICE, SLICE)])


_sc_aggregate = pl.kernel(
    _sc_body,
    out_type=(jax.ShapeDtypeStruct((NC, ACC_ROWS, D_EDGE), jnp.float32),
              jax.ShapeDtypeStruct((NC, ACC_ROWS, D_EDGE), jnp.int16)),
    mesh=plsc.VectorSubcoreMesh(core_axis_name="c", subcore_axis_name="s",
                                num_cores=NC, num_subcores=NS),
    scratch_types=[
        pltpu.VMEM_SHARED((ACC_ROWS, D_EDGE), jnp.float32),
        pltpu.VMEM_SHARED((ACC_ROWS, D_EDGE), jnp.int16),
        pltpu.VMEM((GROUP_EDGES, D_EDGE), jnp.float32),
        pltpu.VMEM((D_EDGE, GROUP_ROWS, IDX_ROW), jnp.float32),
        pltpu.VMEM((GROUP_ROWS, IDX_ROW), jnp.int32),
        pltpu.VMEM((IDX_ROW, D_EDGE), jnp.int16),
    ],
    compiler_params=pltpu.CompilerParams(use_tc_tiling_on_sc=False,
                                         needs_layout_passes=False),
)

BN = 400


def _tc_body(psum_ref, pcnt_ref, nodes_ref, g_ref, out_ref):
    sums = psum_ref[0] + psum_ref[1]
    cnts = (pcnt_ref[0].astype(jnp.float32) +
            pcnt_ref[1].astype(jnp.float32))
    mean = sums / jnp.maximum(cnts, 1.0)
    g = jnp.broadcast_to(g_ref[...], (BN, D_GLOBAL))
    out_ref[...] = jnp.concatenate([mean, nodes_ref[...], g], axis=1)


def kernel(nodes_data, edges_data, global_data, edge_index):
    edges_b = (edges_data.reshape(N_ROWS, IDX_ROW, 2, 8)
               .transpose(2, 0, 3, 1))
    dst_b = (edge_index.astype(jnp.int32)
             .reshape(2, N_ROWS, IDX_ROW).transpose(1, 0, 2))
    zeros_blk = jnp.zeros((SLICE, D_EDGE), jnp.float32)
    zeros_blk_i16 = jnp.zeros((SLICE, D_EDGE), jnp.int16)
    ones_blk = jnp.ones((IDX_ROW, D_EDGE), jnp.int16)

    psum, pcnt = _sc_aggregate(edges_b, dst_b, zeros_blk, zeros_blk_i16,
                               ones_blk)

    out = pl.pallas_call(
        _tc_body,
        grid=(N_NODES // BN,),
        in_specs=[
            pl.BlockSpec((NC, BN, D_EDGE), lambda i: (0, i, 0)),
            pl.BlockSpec((NC, BN, D_EDGE), lambda i: (0, i, 0)),
            pl.BlockSpec((BN, D_NODE), lambda i: (i, 0)),
            pl.BlockSpec((1, D_GLOBAL), lambda i: (0, 0)),
        ],
        out_specs=pl.BlockSpec((BN, D_NODE + 2 * D_EDGE), lambda i: (i, 0)),
        out_shape=jax.ShapeDtypeStruct((N_NODES, D_NODE + 2 * D_EDGE),
                                       jnp.float32),
    )(psum, pcnt, nodes_data, global_data.reshape(1, D_GLOBAL))
    return out


# speedup vs baseline: 6.1451x; 1.7065x over previous
import jax
import jax.numpy as jnp
from jax import lax
from jax.experimental import pallas as pl
from jax.experimental.pallas import tpu as pltpu
from jax.experimental.pallas import tpu_sc as plsc

N_NODES = 50000
N_EDGES = 1600000
D_NODE = 256
D_EDGE = 16
D_GLOBAL = 16

NC = 2
NS = 16
NW = NC * NS

IDX_ROW = 128
N_ROWS = N_EDGES // IDX_ROW
ROWS_BASE = N_ROWS // NW
ROWS_EXTRA = N_ROWS - ROWS_BASE * NW

GROUP_ROWS = 10
GROUP_EDGES = GROUP_ROWS * IDX_ROW
N_GROUPS = ROWS_BASE // GROUP_ROWS

ACC_ROWS = 50048
SLICE = ACC_ROWS // NS


def _sc_body(edges_hbm, dst_hbm, zeros_hbm, zeros16_hbm, ones_hbm,
             psum_hbm, pcnt_hbm, acc_sum, acc_cnt, ebuf, ebuf_t, ibuf,
             ones_v, sem_l, sem_s):
    c = lax.axis_index("c")
    s = lax.axis_index("s")
    w = c * NS + s

    pltpu.sync_copy(zeros_hbm, acc_sum.at[pl.ds(s * SLICE, SLICE)])
    pltpu.sync_copy(zeros16_hbm, acc_cnt.at[pl.ds(s * SLICE, SLICE)])
    pltpu.sync_copy(ones_hbm, ones_v)
    plsc.subcore_barrier()

    row_start = ROWS_BASE * w + jnp.minimum(w, ROWS_EXTRA)

    iota16 = lax.iota(jnp.int32, 16)

    def stage(r0, n_rows):
        cps = [pltpu.async_copy(
                   edges_hbm.at[f // 8, pl.ds(r0, n_rows), f % 8],
                   ebuf_t.at[f, pl.ds(0, n_rows)], sem_l)
               for f in range(D_EDGE)]
        cps.append(pltpu.async_copy(dst_hbm.at[pl.ds(r0, n_rows), 1],
                                    ibuf.at[pl.ds(0, n_rows)], sem_l))
        for cp in cps:
            cp.wait()

        def irow(j, carry):
            for lc in range(IDX_ROW // 16):
                rows = j * IDX_ROW + lc * 16 + iota16
                for f in range(D_EDGE):
                    v = ebuf_t[f, j, pl.ds(lc * 16, 16)]
                    plsc.store_scatter(
                        ebuf, [rows, jnp.full((16,), f, jnp.int32)], v)
            return carry
        lax.fori_loop(0, n_rows, irow, 0)

    def scatter(n_rows):
        cps = []
        for j in range(n_rows):
            cps.append(pltpu.async_copy(ebuf.at[pl.ds(j * IDX_ROW, IDX_ROW)],
                                        acc_sum.at[ibuf.at[j]], sem_s,
                                        add=True))
            cps.append(pltpu.async_copy(ones_v, acc_cnt.at[ibuf.at[j]],
                                        sem_s, add=True))
        for cp in cps:
            cp.wait()

    def group(g, carry):
        r0 = row_start + g * GROUP_ROWS
        stage(r0, GROUP_ROWS)
        scatter(GROUP_ROWS)
        return carry

    lax.fori_loop(0, N_GROUPS, group, 0)

    @pl.when(w < ROWS_EXTRA)
    def _():
        r0 = row_start + ROWS_BASE
        stage(r0, 1)
        scatter(1)

    plsc.subcore_barrier()

    pltpu.sync_copy(acc_sum.at[pl.ds(s * SLICE, SLICE)],
                    psum_hbm.at[c, pl.ds(s * SLICE, SLICE)])
    pltpu.sync_copy(acc_cnt.at[pl.ds(s * SLICE, SLICE)],
                    pcnt_hbm.at[c, pl.ds(s * SLICE, SLICE)])


_sc_aggregate = pl.kernel(
    _sc_body,
    out_type=(jax.ShapeDtypeStruct((NC, ACC_ROWS, D_EDGE), jnp.float32),
              jax.ShapeDtypeStruct((NC, ACC_ROWS, D_EDGE), jnp.int16)),
    mesh=plsc.VectorSubcoreMesh(core_axis_name="c", subcore_axis_name="s",
                                num_cores=NC, num_subcores=NS),
    scratch_types=[
        pltpu.VMEM_SHARED((ACC_ROWS, D_EDGE), jnp.float32),
        pltpu.VMEM_SHARED((ACC_ROWS, D_EDGE), jnp.int16),
        pltpu.VMEM((GROUP_EDGES, D_EDGE), jnp.float32),
        pltpu.VMEM((D_EDGE, GROUP_ROWS, IDX_ROW), jnp.float32),
        pltpu.VMEM((GROUP_ROWS, IDX_ROW), jnp.int32),
        pltpu.VMEM((IDX_ROW, D_EDGE), jnp.int16),
        pltpu.SemaphoreType.DMA,
        pltpu.SemaphoreType.DMA,
    ],
    compiler_params=pltpu.CompilerParams(use_tc_tiling_on_sc=False,
                                         needs_layout_passes=False),
)

BN = 400


def _tc_body(psum_ref, pcnt_ref, nodes_ref, g_ref, out_ref):
    sums = psum_ref[0] + psum_ref[1]
    cnts = (pcnt_ref[0].astype(jnp.float32) +
            pcnt_ref[1].astype(jnp.float32))
    mean = sums / jnp.maximum(cnts, 1.0)
    g = jnp.broadcast_to(g_ref[...], (BN, D_GLOBAL))
    out_ref[...] = jnp.concatenate([mean, nodes_ref[...], g], axis=1)


def kernel(nodes_data, edges_data, global_data, edge_index):
    edges_b = (edges_data.reshape(N_ROWS, IDX_ROW, 2, 8)
               .transpose(2, 0, 3, 1))
    dst_b = (edge_index.astype(jnp.int32)
             .reshape(2, N_ROWS, IDX_ROW).transpose(1, 0, 2))
    zeros_blk = jnp.zeros((SLICE, D_EDGE), jnp.float32)
    zeros_blk_i16 = jnp.zeros((SLICE, D_EDGE), jnp.int16)
    ones_blk = jnp.ones((IDX_ROW, D_EDGE), jnp.int16)

    psum, pcnt = _sc_aggregate(edges_b, dst_b, zeros_blk, zeros_blk_i16,
                               ones_blk)

    out = pl.pallas_call(
        _tc_body,
        grid=(N_NODES // BN,),
        in_specs=[
            pl.BlockSpec((NC, BN, D_EDGE), lambda i: (0, i, 0)),
            pl.BlockSpec((NC, BN, D_EDGE), lambda i: (0, i, 0)),
            pl.BlockSpec((BN, D_NODE), lambda i: (i, 0)),
            pl.BlockSpec((1, D_GLOBAL), lambda i: (0, 0)),
        ],
        out_specs=pl.BlockSpec((BN, D_NODE + 2 * D_EDGE), lambda i: (i, 0)),
        out_shape=jax.ShapeDtypeStruct((N_NODES, D_NODE + 2 * D_EDGE),
                                       jnp.float32),
    )(psum, pcnt, nodes_data, global_data.reshape(1, D_GLOBAL))
    return out


# speedup vs baseline: 7.3413x; 1.1947x over previous
import jax
import jax.numpy as jnp
from jax import lax
from jax.experimental import pallas as pl
from jax.experimental.pallas import tpu as pltpu
from jax.experimental.pallas import tpu_sc as plsc

N_NODES = 50000
N_EDGES = 1600000
D_NODE = 256
D_EDGE = 16
D_GLOBAL = 16

NC = 2
NS = 16
NW = NC * NS

IDX_ROW = 128
N_ROWS = N_EDGES // IDX_ROW
ROWS_BASE = N_ROWS // NW
ROWS_EXTRA = N_ROWS - ROWS_BASE * NW

GROUP_ROWS = 5
GROUP_EDGES = GROUP_ROWS * IDX_ROW
N_GROUPS = ROWS_BASE // GROUP_ROWS

ACC_ROWS = 50048
SLICE = ACC_ROWS // NS


def _sc_body(edges_hbm, dst_hbm, zeros_hbm, zeros16_hbm, ones_hbm,
             psum_hbm, pcnt_hbm, acc_sum, acc_cnt, ebuf, ebuf_t, ibuf,
             ones_v, sem_l, sem_s):
    c = lax.axis_index("c")
    s = lax.axis_index("s")
    w = c * NS + s

    pltpu.sync_copy(zeros_hbm, acc_sum.at[pl.ds(s * SLICE, SLICE)])
    pltpu.sync_copy(zeros16_hbm, acc_cnt.at[pl.ds(s * SLICE, SLICE)])
    pltpu.sync_copy(ones_hbm, ones_v)
    plsc.subcore_barrier()

    row_start = ROWS_BASE * w + jnp.minimum(w, ROWS_EXTRA)

    iota16 = lax.iota(jnp.int32, 16)

    def start_load(r0, n_rows, b):
        for f in range(D_EDGE):
            pltpu.async_copy(edges_hbm.at[f // 8, pl.ds(r0, n_rows), f % 8],
                             ebuf_t.at[b, f, pl.ds(0, n_rows)], sem_l)
        pltpu.async_copy(dst_hbm.at[pl.ds(r0, n_rows), 1],
                         ibuf.at[b, pl.ds(0, n_rows)], sem_l)

    def wait_load(n_rows, b):
        for f in range(D_EDGE):
            pltpu.make_async_copy(
                edges_hbm.at[f // 8, pl.ds(0, n_rows), f % 8],
                ebuf_t.at[b, f, pl.ds(0, n_rows)], sem_l).wait()
        pltpu.make_async_copy(dst_hbm.at[pl.ds(0, n_rows), 1],
                              ibuf.at[b, pl.ds(0, n_rows)], sem_l).wait()

    def interleave(n_rows, b):
        def irow(j, carry):
            for lc in range(IDX_ROW // 16):
                rows = j * IDX_ROW + lc * 16 + iota16
                for f in range(D_EDGE):
                    v = ebuf_t[b, f, j, pl.ds(lc * 16, 16)]
                    plsc.store_scatter(
                        ebuf.at[b], [rows, jnp.full((16,), f, jnp.int32)], v)
            return carry
        lax.fori_loop(0, n_rows, irow, 0)

    def start_scatter(n_rows, b):
        for j in range(n_rows):
            pltpu.async_copy(ebuf.at[b, pl.ds(j * IDX_ROW, IDX_ROW)],
                             acc_sum.at[ibuf.at[b, j]], sem_s, add=True)
            pltpu.async_copy(ones_v, acc_cnt.at[ibuf.at[b, j]],
                             sem_s, add=True)

    def wait_scatter(n_rows, b):
        for j in range(n_rows):
            pltpu.make_async_copy(ebuf.at[b, pl.ds(j * IDX_ROW, IDX_ROW)],
                                  acc_sum.at[ibuf.at[b, j]], sem_s).wait()
            pltpu.make_async_copy(ones_v, acc_cnt.at[ibuf.at[b, j]],
                                  sem_s).wait()

    start_load(row_start, GROUP_ROWS, 0)

    def outer(t, carry):
        for k in range(2):
            g = 2 * t + k
            b = k
            r0 = row_start + g * GROUP_ROWS
            wait_load(GROUP_ROWS, b)
            interleave(GROUP_ROWS, b)

            @pl.when(g + 1 < N_GROUPS)
            def _():
                start_load(r0 + GROUP_ROWS, GROUP_ROWS, 1 - b)

            @pl.when(g >= 1)
            def _():
                wait_scatter(GROUP_ROWS, 1 - b)
            start_scatter(GROUP_ROWS, b)
        return carry

    lax.fori_loop(0, N_GROUPS // 2, outer, 0)
    wait_scatter(GROUP_ROWS, (N_GROUPS - 1) % 2)

    @pl.when(w < ROWS_EXTRA)
    def _():
        r0 = row_start + ROWS_BASE
        start_load(r0, 1, 0)
        wait_load(1, 0)
        interleave(1, 0)
        start_scatter(1, 0)
        wait_scatter(1, 0)

    plsc.subcore_barrier()

    pltpu.sync_copy(acc_sum.at[pl.ds(s * SLICE, SLICE)],
                    psum_hbm.at[c, pl.ds(s * SLICE, SLICE)])
    pltpu.sync_copy(acc_cnt.at[pl.ds(s * SLICE, SLICE)],
                    pcnt_hbm.at[c, pl.ds(s * SLICE, SLICE)])


_sc_aggregate = pl.kernel(
    _sc_body,
    out_type=(jax.ShapeDtypeStruct((NC, ACC_ROWS, D_EDGE), jnp.float32),
              jax.ShapeDtypeStruct((NC, ACC_ROWS, D_EDGE), jnp.int16)),
    mesh=plsc.VectorSubcoreMesh(core_axis_name="c", subcore_axis_name="s",
                                num_cores=NC, num_subcores=NS),
    scratch_types=[
        pltpu.VMEM_SHARED((ACC_ROWS, D_EDGE), jnp.float32),
        pltpu.VMEM_SHARED((ACC_ROWS, D_EDGE), jnp.int16),
        pltpu.VMEM((2, GROUP_EDGES, D_EDGE), jnp.float32),
        pltpu.VMEM((2, D_EDGE, GROUP_ROWS, IDX_ROW), jnp.float32),
        pltpu.VMEM((2, GROUP_ROWS, IDX_ROW), jnp.int32),
        pltpu.VMEM((IDX_ROW, D_EDGE), jnp.int16),
        pltpu.SemaphoreType.DMA,
        pltpu.SemaphoreType.DMA,
    ],
    compiler_params=pltpu.CompilerParams(use_tc_tiling_on_sc=False,
                                         needs_layout_passes=False),
)

BN = 512


def _tc_body(psum_ref, pcnt_ref, nodes_ref, g_ref, out_ref):
    sums = psum_ref[0] + psum_ref[1]
    cnts = (pcnt_ref[0].astype(jnp.float32) +
            pcnt_ref[1].astype(jnp.float32))
    mean = sums / jnp.maximum(cnts, 1.0)
    g = jnp.broadcast_to(g_ref[...].reshape(D_GLOBAL, 1), (D_GLOBAL, BN))
    out_ref[...] = jnp.concatenate([mean.T, nodes_ref[...].T, g], axis=0)


def kernel(nodes_data, edges_data, global_data, edge_index):
    edges_b = (edges_data.reshape(N_ROWS, IDX_ROW, 2, 8)
               .transpose(2, 0, 3, 1))
    dst_b = (edge_index.astype(jnp.int32)
             .reshape(2, N_ROWS, IDX_ROW).transpose(1, 0, 2))
    zeros_blk = jnp.zeros((SLICE, D_EDGE), jnp.float32)
    zeros_blk_i16 = jnp.zeros((SLICE, D_EDGE), jnp.int16)
    ones_blk = jnp.ones((IDX_ROW, D_EDGE), jnp.int16)

    psum, pcnt = _sc_aggregate(edges_b, dst_b, zeros_blk, zeros_blk_i16,
                               ones_blk)

    out = pl.pallas_call(
        _tc_body,
        grid=(pl.cdiv(N_NODES, BN),),
        in_specs=[
            pl.BlockSpec((NC, BN, D_EDGE), lambda i: (0, i, 0)),
            pl.BlockSpec((NC, BN, D_EDGE), lambda i: (0, i, 0)),
            pl.BlockSpec((BN, D_NODE), lambda i: (i, 0)),
            pl.BlockSpec((1, D_GLOBAL), lambda i: (0, 0)),
        ],
        out_specs=pl.BlockSpec((D_NODE + 2 * D_EDGE, BN), lambda i: (0, i)),
        out_shape=jax.ShapeDtypeStruct((D_NODE + 2 * D_EDGE, N_NODES),
                                       jnp.float32),
    )(psum, pcnt, nodes_data, global_data.reshape(1, D_GLOBAL))
    return out.T
